# Initial kernel scaffold; baseline (speedup 1.0000x reference)
import jax
import jax.numpy as jnp
from jax.experimental import pallas as pl


def kernel(x, x_adj_s, x_adj_t, t_W1, t_W2, t_W3, s_W1, s_W2, s_W3, dec1_W, dec1_b, dec2_W, dec2_b, dec3_W, dec3_b, fc_W, fc_b):
    raise NotImplementedError("write your pallas kernel here")


# speedup vs baseline: 1.4499x; 1.4499x over previous
import jax
import jax.numpy as jnp
from jax.experimental import pallas as pl
from jax.experimental.pallas import tpu as pltpu


def _dot(a, b):
    return jax.lax.dot_general(a, b, (((1,), (0,)), ((), ())),
                               preferred_element_type=jnp.float32)


def _dot_tn(a, b):
    return jax.lax.dot_general(a, b, (((0,), (0,)), ((), ())),
                               preferred_element_type=jnp.float32)


def _dot_nt(a, b):
    return jax.lax.dot_general(a, b, (((1,), (1,)), ((), ())),
                               preferred_element_type=jnp.float32)


def _body(coef_ref, x_ref, adj_s_ref, adj_t_ref, tw1_ref, tw2_ref, tw3_ref,
          sw1_ref, sw2_ref, sw3_ref, fcw_ref, fcb_ref, out_ref):
    x = x_ref[...]
    adj_t = adj_t_ref[...]
    adj_s = adj_s_ref[...]

    h = jnp.maximum(_dot(adj_t, _dot(x, tw1_ref[...])), 0.0)
    h = jnp.maximum(_dot(adj_t, _dot(h, tw2_ref[...])), 0.0)
    x_t = jnp.maximum(_dot(_dot(adj_t, h), tw3_ref[...]), 0.0)

    g = jnp.maximum(_dot(adj_s, _dot_tn(x, sw1_ref[...])), 0.0)
    g = jnp.maximum(_dot(adj_s, _dot(g, sw2_ref[...])), 0.0)
    q = _dot(adj_s, g)
    x_st = jnp.maximum(
        jax.lax.dot_general(sw3_ref[...], q, (((0,), (1,)), ((), ())),
                            preferred_element_type=jnp.float32), 0.0)

    a_s = coef_ref[0]
    a_t = coef_ref[1]
    b0 = coef_ref[2]
    fused = a_s * x_st + a_t * x_t + b0

    out_ref[...] = _dot_nt(fused, fcw_ref[...]) + fcb_ref[...]


def kernel(x, x_adj_s, x_adj_t, t_W1, t_W2, t_W3, s_W1, s_W2, s_W3,
           dec1_W, dec1_b, dec2_W, dec2_b, dec3_W, dec3_b, fc_W, fc_b):
    T, Kd = x.shape

    m = dec1_W @ dec2_W @ dec3_W
    b_eff = (dec1_b @ dec2_W + dec2_b) @ dec3_W + dec3_b
    coef = jnp.concatenate([m[:, 0], b_eff]).astype(jnp.float32)

    vmem = pl.BlockSpec(memory_space=pltpu.VMEM)
    out = pl.pallas_call(
        _body,
        out_shape=jax.ShapeDtypeStruct((T, Kd), jnp.float32),
        in_specs=[pl.BlockSpec(memory_space=pltpu.SMEM)] + [vmem] * 11,
        out_specs=vmem,
    )(coef, x, x_adj_s, x_adj_t,
      t_W1[0], t_W2[0], t_W3[0], s_W1[0], s_W2[0], s_W3[0],
      fc_W, fc_b.reshape(1, Kd))
    return out


# speedup vs baseline: 1.6165x; 1.1149x over previous
import jax
import jax.numpy as jnp
from jax.experimental import pallas as pl
from jax.experimental.pallas import tpu as pltpu


def _dot(a, b):
    return jax.lax.dot_general(a, b, (((1,), (0,)), ((), ())),
                               preferred_element_type=jnp.float32)


def _dot_tn(a, b):
    return jax.lax.dot_general(a, b, (((0,), (0,)), ((), ())),
                               preferred_element_type=jnp.float32)


def _dot_nt(a, b):
    return jax.lax.dot_general(a, b, (((1,), (1,)), ((), ())),
                               preferred_element_type=jnp.float32)


def _body(coef_ref, x_ref, adj_s_ref, adj_t_ref, tw1_ref, tw2_ref, tw3_ref,
          sw1_ref, sw2_ref, sw3_ref, fcw_ref, fcb_ref, out_ref):
    x = x_ref[...]
    adj_t = adj_t_ref[...]
    adj_s = adj_s_ref[...]


    t1 = jax.lax.dot_general(tw1_ref[...], x, (((0,), (1,)), ((), ())),
                             preferred_element_type=jnp.float32)
    h = jnp.maximum(_dot_nt(t1, adj_t), 0.0)
    h = jnp.maximum(_dot_nt(_dot_tn(tw2_ref[...], h), adj_t), 0.0)
    r = _dot_nt(h, adj_t)
    x_t = jnp.maximum(_dot_tn(r, tw3_ref[...]), 0.0)

    s1 = _dot_tn(sw1_ref[...], x)
    g = jnp.maximum(_dot_nt(s1, adj_s), 0.0)
    g = jnp.maximum(_dot_nt(_dot_tn(sw2_ref[...], g), adj_s), 0.0)
    q = _dot_nt(g, adj_s)
    x_st = jnp.maximum(_dot_tn(sw3_ref[...], q), 0.0)

    a_s = coef_ref[0]
    a_t = coef_ref[1]
    b0 = coef_ref[2]
    fused = a_s * x_st + a_t * x_t + b0

    out_ref[...] = _dot_nt(fused, fcw_ref[...]) + fcb_ref[...]


def kernel(x, x_adj_s, x_adj_t, t_W1, t_W2, t_W3, s_W1, s_W2, s_W3,
           dec1_W, dec1_b, dec2_W, dec2_b, dec3_W, dec3_b, fc_W, fc_b):
    T, Kd = x.shape

    m = dec1_W @ dec2_W @ dec3_W
    b_eff = (dec1_b @ dec2_W + dec2_b) @ dec3_W + dec3_b
    coef = jnp.concatenate([m[:, 0], b_eff]).astype(jnp.float32)

    vmem = pl.BlockSpec(memory_space=pltpu.VMEM)
    out = pl.pallas_call(
        _body,
        out_shape=jax.ShapeDtypeStruct((T, Kd), jnp.float32),
        in_specs=[pl.BlockSpec(memory_space=pltpu.SMEM)] + [vmem] * 11,
        out_specs=vmem,
    )(coef, x, x_adj_s, x_adj_t,
      t_W1[0], t_W2[0], t_W3[0], s_W1[0], s_W2[0], s_W3[0],
      fc_W, fc_b.reshape(1, Kd))
    return out


# speedup vs baseline: 1.7079x; 1.0566x over previous
import jax
import jax.numpy as jnp
from jax.experimental import pallas as pl
from jax.experimental.pallas import tpu as pltpu


def _dot(a, b):
    return jax.lax.dot_general(a, b, (((1,), (0,)), ((), ())),
                               preferred_element_type=jnp.float32)


def _dot_tn(a, b):
    return jax.lax.dot_general(a, b, (((0,), (0,)), ((), ())),
                               preferred_element_type=jnp.float32)


def _dot_nt(a, b):
    return jax.lax.dot_general(a, b, (((1,), (1,)), ((), ())),
                               preferred_element_type=jnp.float32)


def _body(x_ref, adj_s_ref, adj_t_ref, tw1_ref, tw2_ref, tw3_ref,
          sw1_ref, sw2_ref, sw3_ref, d1w_ref, d1b_ref, d2w_ref, d2b_ref,
          d3w_ref, d3b_ref, fcw_ref, fcb_ref, out_ref):
    x = x_ref[...]
    adj_t = adj_t_ref[...]
    adj_s = adj_s_ref[...]

    m23 = _dot(d2w_ref[...], d3w_ref[...])
    m = _dot(d1w_ref[...], m23)
    b_eff = _dot(_dot(d1b_ref[...], d2w_ref[...]) + d2b_ref[...],
                 d3w_ref[...]) + d3b_ref[...]
    a_s = m[0, 0]
    a_t = m[1, 0]
    b0 = b_eff[0, 0]


    t1 = jax.lax.dot_general(tw1_ref[...], x, (((0,), (1,)), ((), ())),
                             preferred_element_type=jnp.float32)
    h = jnp.maximum(_dot_nt(t1, adj_t), 0.0)
    h = jnp.maximum(_dot_nt(_dot_tn(tw2_ref[...], h), adj_t), 0.0)
    r = _dot_nt(h, adj_t)
    x_t = jnp.maximum(_dot_tn(r, tw3_ref[...]), 0.0)

    s1 = _dot_tn(sw1_ref[...], x)
    g = jnp.maximum(_dot_nt(s1, adj_s), 0.0)
    g = jnp.maximum(_dot_nt(_dot_tn(sw2_ref[...], g), adj_s), 0.0)
    q = _dot_nt(g, adj_s)
    x_st = jnp.maximum(_dot_tn(sw3_ref[...], q), 0.0)

    fused = a_s * x_st + a_t * x_t + b0

    out_ref[...] = _dot_nt(fused, fcw_ref[...]) + fcb_ref[...]


def kernel(x, x_adj_s, x_adj_t, t_W1, t_W2, t_W3, s_W1, s_W2, s_W3,
           dec1_W, dec1_b, dec2_W, dec2_b, dec3_W, dec3_b, fc_W, fc_b):
    T, Kd = x.shape

    vmem = pl.BlockSpec(memory_space=pltpu.VMEM)
    out = pl.pallas_call(
        _body,
        out_shape=jax.ShapeDtypeStruct((T, Kd), jnp.float32),
        in_specs=[vmem] * 17,
        out_specs=vmem,
    )(x, x_adj_s, x_adj_t,
      t_W1[0], t_W2[0], t_W3[0], s_W1[0], s_W2[0], s_W3[0],
      dec1_W, dec1_b.reshape(1, 8), dec2_W, dec2_b.reshape(1, 4),
      dec3_W, dec3_b.reshape(1, 1), fc_W, fc_b.reshape(1, Kd))
    return out
